# baseline (speedup 1.0000x reference)
import functools

import jax
import jax.numpy as jnp
from jax import lax
from jax.experimental import pallas as pl
from jax.experimental.pallas import tpu as pltpu
from jax.experimental.pallas import tpu_sc as plsc

N_NODES = 10000
N_PAD = 10240
N_EDGES = 320000
D = 128
NC = 2
NS = 16
NW = NC * NS
EPW = N_EDGES // NW
CHUNK = 80
NJ = EPW // CHUNK
RPT = N_PAD // NS
ZR = 16

SCH = 128
EPW_P = 10240
E_PAD = NW * EPW_P
NSJ = EPW_P // SCH
SNG = 4
SGC = NSJ // SNG
NSLOT = 2

_mesh = plsc.VectorSubcoreMesh(core_axis_name="c", subcore_axis_name="s")


def _fill_vec(ref, n16, val):
    def body(i, _):
        ref[pl.ds(i * 16, 16)] = jnp.full((16,), val, jnp.float32)
        return 0
    lax.fori_loop(0, n16, body, 0)


@functools.partial(
    pl.kernel,
    out_type=jax.ShapeDtypeStruct((NC, N_PAD), jnp.float32),
    mesh=_mesh,
    scratch_types=[
        pltpu.VMEM_SHARED((N_PAD,), jnp.float32),
        pltpu.VMEM((SGC, SCH), jnp.int32),
        pltpu.VMEM((SCH,), jnp.float32),
        pltpu.VMEM((RPT,), jnp.float32),
    ],
)
def _sc_degree(dst_hbm, out_hbm, acc, dstv, onesv, zerov):
    c = lax.axis_index("c")
    s = lax.axis_index("s")
    wid = c * NS + s
    _fill_vec(onesv, SCH // 16, 1.0)
    _fill_vec(zerov, RPT // 16, 0.0)
    pltpu.sync_copy(zerov, acc.at[pl.ds(s * RPT, RPT)])
    plsc.subcore_barrier()

    def group(g, _):
        pltpu.sync_copy(dst_hbm.at[wid, g], dstv)

        def step(j, _):
            pltpu.sync_copy(onesv, acc.at[dstv.at[j]], add=True)
            return 0
        lax.fori_loop(0, SGC, step, 0)
        return 0
    lax.fori_loop(0, SNG, group, 0)
    plsc.subcore_barrier()
    pltpu.sync_copy(acc.at[pl.ds(s * RPT, RPT)],
                    out_hbm.at[c, pl.ds(s * RPT, RPT)])


@functools.partial(
    pl.kernel,
    out_type=jax.ShapeDtypeStruct((NC, N_PAD, D), jnp.float32),
    mesh=_mesh,
    scratch_types=[
        pltpu.VMEM_SHARED((N_PAD, D), jnp.float32),
        pltpu.VMEM((SGC, SCH), jnp.int32),
        pltpu.VMEM((SGC, SCH), jnp.int32),
        [pltpu.VMEM((SCH, D), jnp.float32) for _ in range(NSLOT)],
        pltpu.VMEM((ZR, D), jnp.float32),
        [pltpu.SemaphoreType.DMA for _ in range(NSLOT)],
        [pltpu.SemaphoreType.DMA for _ in range(NSLOT)],
    ],
)
def _sc_scatter(xw_hbm, src_hbm, dst_hbm, out_hbm,
                acc, srcv, dstv, rows, zerov, sg, ss):
    c = lax.axis_index("c")
    s = lax.axis_index("s")
    wid = c * NS + s

    def start_g(u, k):
        pltpu.async_copy(xw_hbm.at[srcv.at[k]], rows[u], sg[u])

    def wait_g(u):
        pltpu.make_async_copy(xw_hbm.at[srcv.at[0]], rows[u], sg[u]).wait()

    def start_s(u, k):
        pltpu.async_copy(rows[u], acc.at[dstv.at[k]], ss[u], add=True)

    def wait_s(u):
        pltpu.make_async_copy(rows[u], acc.at[dstv.at[0]], ss[u]).wait()

    def quads(b, _):
        for u in range(NSLOT):
            wait_g(u)
            start_s(u, b * NSLOT + u)
        for u in range(NSLOT):
            wait_s(u)
            start_g(u, b * NSLOT + NSLOT + u)
        return 0

    def run_group(g, first):
        pltpu.sync_copy(src_hbm.at[wid, g], srcv)
        pltpu.sync_copy(dst_hbm.at[wid, g], dstv)
        for u in range(NSLOT):
            if not first:
                wait_s(u)
            start_g(u, u)
        if first:
            def zbody(k, _):
                zerov[k // 8, pl.ds((k % 8) * 16, 16)] = \
                    jnp.zeros((16,), jnp.float32)
                return 0
            lax.fori_loop(0, ZR * 8, zbody, 0)

            def zcopy(t, _):
                pltpu.sync_copy(zerov, acc.at[pl.ds(s * RPT + t * ZR, ZR)])
                return 0
            lax.fori_loop(0, RPT // ZR, zcopy, 0)
            plsc.subcore_barrier()
        lax.fori_loop(0, SGC // NSLOT - 1, quads, 0)
        for u in range(NSLOT):
            wait_g(u)
            start_s(u, SGC - NSLOT + u)

    run_group(0, True)

    def group(g, _):
        run_group(g, False)
        return 0
    lax.fori_loop(1, SNG, group, 0)
    for u in range(NSLOT):
        wait_s(u)

    plsc.subcore_barrier()
    pltpu.sync_copy(acc.at[pl.ds(s * RPT, RPT)],
                    out_hbm.at[c, pl.ds(s * RPT, RPT)])


_RB = 2000
_GRID = N_NODES // _RB

_rb = pl.BlockSpec((_RB, D), lambda i: (i, 0))
_rb1 = pl.BlockSpec((_RB, 1), lambda i: (i, 0))
_p0 = pl.BlockSpec((1, _RB, D), lambda i: (0, i, 0))
_p1 = pl.BlockSpec((1, _RB, D), lambda i: (1, i, 0))
_d0 = pl.BlockSpec((1, _RB, 1), lambda i: (0, i, 0))
_d1 = pl.BlockSpec((1, _RB, 1), lambda i: (1, i, 0))
_wfull = pl.BlockSpec((D, D), lambda i: (0, 0))
_bfull = pl.BlockSpec((1, D), lambda i: (0, 0))


def _tc_prep1_body(x_ref, w_ref, d0_ref, d1_ref, xwp_ref, dis_ref):
    dis = lax.rsqrt(d0_ref[0] + d1_ref[0] + 1.0)
    xwp_ref[...] = jnp.dot(x_ref[...], w_ref[...],
                           preferred_element_type=jnp.float32) * dis
    dis_ref[...] = dis


def _tc_prep1(x, W1, degP):
    return pl.pallas_call(
        _tc_prep1_body,
        grid=(_GRID,),
        in_specs=[_rb, _wfull, _d0, _d1],
        out_specs=[_rb, _rb1],
        out_shape=[
            jax.ShapeDtypeStruct((N_NODES, D), jnp.float32),
            jax.ShapeDtypeStruct((N_NODES, 1), jnp.float32),
        ],
    )(x, W1, degP, degP)


def _tc_mid_body(p_ref, q_ref, xwp_ref, dis_ref, b_ref, w_ref, out_ref):
    dis = dis_ref[...]
    h = jax.nn.relu(dis * (p_ref[0] + q_ref[0] + xwp_ref[...]) + b_ref[...])
    out_ref[...] = jnp.dot(h, w_ref[...],
                           preferred_element_type=jnp.float32) * dis


def _tc_mid(P, xwp, dis, b1, W2):
    return pl.pallas_call(
        _tc_mid_body,
        grid=(_GRID,),
        in_specs=[_p0, _p1, _rb, _rb1, _bfull, _wfull],
        out_specs=_rb,
        out_shape=jax.ShapeDtypeStruct((N_NODES, D), jnp.float32),
    )(P, P, xwp, dis, b1, W2)


def _tc_final_body(p_ref, q_ref, xwp_ref, dis_ref, b_ref, out_ref):
    out_ref[...] = dis_ref[...] * (p_ref[0] + q_ref[0] + xwp_ref[...]) \
        + b_ref[...]


def _tc_final(P, xwp, dis, b2):
    return pl.pallas_call(
        _tc_final_body,
        grid=(_GRID,),
        in_specs=[_p0, _p1, _rb, _rb1, _bfull],
        out_specs=_rb,
        out_shape=jax.ShapeDtypeStruct((N_NODES, D), jnp.float32),
    )(P, P, xwp, dis, b2)


def kernel(x, edge_index, W1, b1, W2, b2):
    pad_n = E_PAD - N_EDGES
    ar = jnp.arange(pad_n, dtype=jnp.int32)
    pad_src = ((ar * 13) % N_NODES).reshape(pad_n // SCH, SCH)
    pad_dst = (N_NODES + ar % (N_PAD - N_NODES)).reshape(pad_n // SCH, SCH)
    er = edge_index.astype(jnp.int32).reshape(2, N_EDGES // SCH, SCH)
    srcR = jnp.concatenate([er[0], pad_src]).reshape(NW, SNG, SGC, SCH)
    dstR = jnp.concatenate([er[1], pad_dst]).reshape(NW, SNG, SGC, SCH)
    b1r = b1.reshape(1, D)
    b2r = b2.reshape(1, D)

    degP = _sc_degree(dstR).reshape(NC, N_PAD, 1)
    xw1p, dis = _tc_prep1(x, W1, degP)
    P1 = _sc_scatter(xw1p, srcR, dstR)
    xw2p = _tc_mid(P1, xw1p, dis, b1r, W2)
    P2 = _sc_scatter(xw2p, srcR, dstR)
    return _tc_final(P2, xw2p, dis, b2r)


# speedup vs baseline: 1.1768x; 1.1768x over previous
import functools

import jax
import jax.numpy as jnp
from jax import lax
from jax.experimental import pallas as pl
from jax.experimental.pallas import tpu as pltpu
from jax.experimental.pallas import tpu_sc as plsc

N_NODES = 10000
N_PAD = 10240
N_EDGES = 320000
D = 128
NC = 2
NS = 16
NW = NC * NS
EPW = N_EDGES // NW
CHUNK = 80
NJ = EPW // CHUNK
RPT = N_PAD // NS
ZR = 16

SCH = 64
EPW_P = 10240
E_PAD = NW * EPW_P
NSJ = EPW_P // SCH
SNG = 4
SGC = NSJ // SNG
NSLOT = 4

DCH = 128
DNG = 4
DGC = EPW_P // (DCH * DNG)

_mesh = plsc.VectorSubcoreMesh(core_axis_name="c", subcore_axis_name="s")


def _fill_vec(ref, n16, val):
    def body(i, _):
        ref[pl.ds(i * 16, 16)] = jnp.full((16,), val, jnp.float32)
        return 0
    lax.fori_loop(0, n16, body, 0)


@functools.partial(
    pl.kernel,
    out_type=jax.ShapeDtypeStruct((NC, N_PAD), jnp.float32),
    mesh=_mesh,
    scratch_types=[
        pltpu.VMEM_SHARED((N_PAD,), jnp.float32),
        pltpu.VMEM((DGC, DCH), jnp.int32),
        pltpu.VMEM((DCH,), jnp.float32),
        pltpu.VMEM((RPT,), jnp.float32),
    ],
)
def _sc_degree(dst_hbm, out_hbm, acc, dstv, onesv, zerov):
    c = lax.axis_index("c")
    s = lax.axis_index("s")
    wid = c * NS + s
    _fill_vec(onesv, DCH // 16, 1.0)
    _fill_vec(zerov, RPT // 16, 0.0)
    pltpu.sync_copy(zerov, acc.at[pl.ds(s * RPT, RPT)])
    plsc.subcore_barrier()

    def group(g, _):
        pltpu.sync_copy(dst_hbm.at[wid, g], dstv)

        def step(j, _):
            pltpu.sync_copy(onesv, acc.at[dstv.at[j]], add=True)
            return 0
        lax.fori_loop(0, DGC, step, 0)
        return 0
    lax.fori_loop(0, DNG, group, 0)
    plsc.subcore_barrier()
    pltpu.sync_copy(acc.at[pl.ds(s * RPT, RPT)],
                    out_hbm.at[c, pl.ds(s * RPT, RPT)])


@functools.partial(
    pl.kernel,
    out_type=jax.ShapeDtypeStruct((NC, N_PAD, D), jnp.float32),
    mesh=_mesh,
    scratch_types=[
        pltpu.VMEM_SHARED((N_PAD, D), jnp.float32),
        pltpu.VMEM((SGC, SCH), jnp.int32),
        pltpu.VMEM((SGC, SCH), jnp.int32),
        [pltpu.VMEM((SCH, D), jnp.float32) for _ in range(NSLOT)],
        pltpu.VMEM((ZR, D), jnp.float32),
        [pltpu.SemaphoreType.DMA for _ in range(NSLOT)],
        [pltpu.SemaphoreType.DMA for _ in range(NSLOT)],
    ],
)
def _sc_scatter(xw_hbm, src_hbm, dst_hbm, out_hbm,
                acc, srcv, dstv, rows, zerov, sg, ss):
    c = lax.axis_index("c")
    s = lax.axis_index("s")
    wid = c * NS + s

    def start_g(u, k):
        pltpu.async_copy(xw_hbm.at[srcv.at[k]], rows[u], sg[u])

    def wait_g(u):
        pltpu.make_async_copy(xw_hbm.at[srcv.at[0]], rows[u], sg[u]).wait()

    def start_s(u, k):
        pltpu.async_copy(rows[u], acc.at[dstv.at[k]], ss[u], add=True)

    def wait_s(u):
        pltpu.make_async_copy(rows[u], acc.at[dstv.at[0]], ss[u]).wait()

    def quads(b, _):
        for u in range(NSLOT):
            wait_g(u)
            start_s(u, b * NSLOT + u)
        for u in range(NSLOT):
            wait_s(u)
            start_g(u, b * NSLOT + NSLOT + u)
        return 0

    def run_group(g, first):
        pltpu.sync_copy(src_hbm.at[wid, g], srcv)
        pltpu.sync_copy(dst_hbm.at[wid, g], dstv)
        for u in range(NSLOT):
            if not first:
                wait_s(u)
            start_g(u, u)
        if first:
            def zbody(k, _):
                zerov[k // 8, pl.ds((k % 8) * 16, 16)] = \
                    jnp.zeros((16,), jnp.float32)
                return 0
            lax.fori_loop(0, ZR * 8, zbody, 0)

            def zcopy(t, _):
                pltpu.sync_copy(zerov, acc.at[pl.ds(s * RPT + t * ZR, ZR)])
                return 0
            lax.fori_loop(0, RPT // ZR, zcopy, 0)
            plsc.subcore_barrier()
        lax.fori_loop(0, SGC // NSLOT - 1, quads, 0)
        for u in range(NSLOT):
            wait_g(u)
            start_s(u, SGC - NSLOT + u)

    run_group(0, True)

    def group(g, _):
        run_group(g, False)
        return 0
    lax.fori_loop(1, SNG, group, 0)
    for u in range(NSLOT):
        wait_s(u)

    plsc.subcore_barrier()
    pltpu.sync_copy(acc.at[pl.ds(s * RPT, RPT)],
                    out_hbm.at[c, pl.ds(s * RPT, RPT)])


_RB = 2000
_GRID = N_NODES // _RB

_rb = pl.BlockSpec((_RB, D), lambda i: (i, 0))
_rb1 = pl.BlockSpec((_RB, 1), lambda i: (i, 0))
_p0 = pl.BlockSpec((1, _RB, D), lambda i: (0, i, 0))
_p1 = pl.BlockSpec((1, _RB, D), lambda i: (1, i, 0))
_d0 = pl.BlockSpec((1, _RB, 1), lambda i: (0, i, 0))
_d1 = pl.BlockSpec((1, _RB, 1), lambda i: (1, i, 0))
_wfull = pl.BlockSpec((D, D), lambda i: (0, 0))
_bfull = pl.BlockSpec((1, D), lambda i: (0, 0))


def _tc_prep1_body(x_ref, w_ref, d0_ref, d1_ref, xwp_ref, dis_ref):
    dis = lax.rsqrt(d0_ref[0] + d1_ref[0] + 1.0)
    xwp_ref[...] = jnp.dot(x_ref[...], w_ref[...],
                           preferred_element_type=jnp.float32) * dis
    dis_ref[...] = dis


def _tc_prep1(x, W1, degP):
    return pl.pallas_call(
        _tc_prep1_body,
        grid=(_GRID,),
        in_specs=[_rb, _wfull, _d0, _d1],
        out_specs=[_rb, _rb1],
        out_shape=[
            jax.ShapeDtypeStruct((N_NODES, D), jnp.float32),
            jax.ShapeDtypeStruct((N_NODES, 1), jnp.float32),
        ],
    )(x, W1, degP, degP)


def _tc_mid_body(p_ref, q_ref, xwp_ref, dis_ref, b_ref, w_ref, out_ref):
    dis = dis_ref[...]
    h = jax.nn.relu(dis * (p_ref[0] + q_ref[0] + xwp_ref[...]) + b_ref[...])
    out_ref[...] = jnp.dot(h, w_ref[...],
                           preferred_element_type=jnp.float32) * dis


def _tc_mid(P, xwp, dis, b1, W2):
    return pl.pallas_call(
        _tc_mid_body,
        grid=(_GRID,),
        in_specs=[_p0, _p1, _rb, _rb1, _bfull, _wfull],
        out_specs=_rb,
        out_shape=jax.ShapeDtypeStruct((N_NODES, D), jnp.float32),
    )(P, P, xwp, dis, b1, W2)


def _tc_final_body(p_ref, q_ref, xwp_ref, dis_ref, b_ref, out_ref):
    out_ref[...] = dis_ref[...] * (p_ref[0] + q_ref[0] + xwp_ref[...]) \
        + b_ref[...]


def _tc_final(P, xwp, dis, b2):
    return pl.pallas_call(
        _tc_final_body,
        grid=(_GRID,),
        in_specs=[_p0, _p1, _rb, _rb1, _bfull],
        out_specs=_rb,
        out_shape=jax.ShapeDtypeStruct((N_NODES, D), jnp.float32),
    )(P, P, xwp, dis, b2)


def kernel(x, edge_index, W1, b1, W2, b2):
    pad_n = E_PAD - N_EDGES
    ar = jnp.arange(pad_n, dtype=jnp.int32)
    pad_src = ((ar * 13) % N_NODES).reshape(pad_n // DCH, DCH)
    pad_dst = (N_NODES + ar % (N_PAD - N_NODES)).reshape(pad_n // DCH, DCH)
    er = edge_index.astype(jnp.int32).reshape(2, N_EDGES // DCH, DCH)
    srcR = jnp.concatenate([er[0], pad_src]).reshape(NW, SNG, SGC, SCH)
    dstP = jnp.concatenate([er[1], pad_dst])
    dstR = dstP.reshape(NW, SNG, SGC, SCH)
    dstF = dstP.reshape(NW, DNG, DGC, DCH)
    b1r = b1.reshape(1, D)
    b2r = b2.reshape(1, D)

    degP = _sc_degree(dstF).reshape(NC, N_PAD, 1)
    xw1p, dis = _tc_prep1(x, W1, degP)
    P1 = _sc_scatter(xw1p, srcR, dstR)
    xw2p = _tc_mid(P1, xw1p, dis, b1r, W2)
    P2 = _sc_scatter(xw2p, srcR, dstR)
    return _tc_final(P2, xw2p, dis, b2r)
